# baseline (speedup 1.0000x reference)
import numpy as np
import jax
import jax.numpy as jnp
from jax import lax
from jax.experimental import pallas as pl
from jax.experimental.pallas import tpu as pltpu
from jax.experimental.pallas import tpu_sc as plsc

_B = 4096
_F = 256
_M = 128
_NPAIR = _M * (_M + 1) // 2
_OUT = _M + _NPAIR
_NW = 32
_RPW = _B // _NW
_L = 16
_NLG = _RPW // _L
_CHW = 64
_NCH = _OUT // _CHW
_XT = (_M + 1) * _RPW + 4 * _RPW


def _part_tables():
    off = lambda i: i * _M - i * (i - 1) // 2
    seg = np.zeros(_NPAIR, np.int32)
    for i in range(_M):
        seg[off(i):off(i) + _M - i] = i
    ii = np.empty(_OUT, np.int32)
    jj = np.empty(_OUT, np.int32)
    ii[:_M] = _M
    jj[:_M] = np.arange(_M)
    for c in range(_M, _OUT):
        p = c - _M
        i = int(seg[p])
        ii[c] = i
        jj[c] = i + (p - off(i))
    c0l, n, ia, jb = [], [], [], []
    for c in range(_OUT):
        if (c == 0 or ii[c] != ii[c - 1] or jj[c] != jj[c - 1] + 1
                or c % _CHW == 0):
            c0l.append(c % _CHW)
            n.append(0)
            ia.append(int(ii[c]) * _RPW)
            jb.append(int(jj[c]) * _RPW)
        n[-1] += 1
    starts = np.cumsum([0] + n[:-1])
    cpi = np.searchsorted(starts, np.arange(_NCH) * _CHW, side="left")
    cpi = np.append(cpi, len(n)).astype(np.int32)
    assert sum(n) == _OUT and max(n) <= _CHW
    return (np.asarray(c0l, np.int32), np.asarray(n, np.int32),
            np.asarray(ia, np.int32), np.asarray(jb, np.int32), cpi)


_C0L, _N, _IA, _JB, _CPI = _part_tables()
_NP = _C0L.size
_TBL = np.concatenate([_C0L, _N, _IA, _JB, _CPI])
_NTW = ((_TBL.size + 15) // 16) * 16


def _stage_midx(mk_v, mi_v):
    off = jnp.int32(0)
    lanes = lax.iota(jnp.int32, _L)
    for c in range(_F // _L):
        chunk = mk_v[pl.ds(_L * c, _L)]
        m = chunk > 0
        plsc.store_compressed(mi_v.at[pl.ds(off, _L)], lanes + _L * c, mask=m)
        off = off + plsc.all_reduce_population_count(m)[0]


def _stage_smem(tbl_v, tbl_s):
    for t in range(_NTW // _L):
        v = tbl_v[pl.ds(_L * t, _L)]
        for k in range(_L):
            if _L * t + k < _TBL.size:
                tbl_s[_L * t + k] = v[k]


def _stage_xmt(xblk, mi_v, xmt):
    ones = jnp.full((_L,), 1.0, jnp.float32)
    for lg in range(_NLG):
        xmt[pl.ds(_M * _RPW + _L * lg, _L)] = ones
    lanes = lax.iota(jnp.int32, _L)
    brow = [lanes + _L * lg for lg in range(_NLG)]

    @pl.loop(0, _M // _L)
    def _f16(f16):
        mi = mi_v[pl.ds(_L * f16, _L)]
        base = f16 * (_L * _RPW)
        pend = {}
        for k in range(_L + 1):
            if k < _L:
                mib = jnp.full((_L,), mi[k])
                pend[k] = [plsc.load_gather(xblk, [brow[lg], mib])
                           for lg in range(_NLG)]
            if k >= 1:
                vals = pend.pop(k - 1)
                for lg in range(_NLG):
                    xmt[pl.ds(base + _RPW * (k - 1) + _L * lg, _L)] = vals[lg]


def _chunk_body(ch, wait_pred, xmt, tbl_s, buf, sem, out_hbm, colbase):
    dst = out_hbm.at[pl.ds(ch * _CHW, _CHW), pl.ds(colbase, _RPW)]
    src = buf.at[pl.ds(0, _CHW), :]

    def _waits():
        pltpu.make_async_copy(src, dst, sem).wait()

    if wait_pred is None:
        _waits()
    else:
        pl.when(wait_pred)(_waits)

    plo = tbl_s[4 * _NP + ch]
    phi = tbl_s[4 * _NP + ch + 1]

    @pl.loop(plo, phi)
    def _part(pp):
        c0 = tbl_s[pp]
        npart = tbl_s[_NP + pp]
        ia = tbl_s[2 * _NP + pp]
        jb = tbl_s[3 * _NP + pp]
        va = [xmt[pl.ds(ia + _L * lg, _L)] for lg in range(_NLG)]

        @plsc.parallel_loop(0, (npart + 3) >> 2, unroll=4)
        def _colq(kq):
            k0 = kq * 4
            for u in range(4):
                jaddr = jb + (k0 + u) * _RPW
                vb = [xmt[pl.ds(jaddr + _L * lg, _L)] for lg in range(_NLG)]
                for lg in range(_NLG):
                    buf[c0 + k0 + u, pl.ds(_L * lg, _L)] = va[lg] * vb[lg]

    pltpu.async_copy(src, dst, sem)


def _body(x_hbm, m_hbm, tbl_hbm, out_hbm, xblk, mk_v, mi_v, tbl_v, xmt, bufa,
          bufb, tbl_s, sema, semb):
    wid = lax.axis_index("s") * 2 + lax.axis_index("c")
    base = wid * _RPW
    pltpu.sync_copy(x_hbm.at[pl.ds(base, _RPW), :], xblk)
    pltpu.sync_copy(m_hbm, mk_v)
    pltpu.sync_copy(tbl_hbm, tbl_v)
    _stage_midx(mk_v, mi_v)
    _stage_smem(tbl_v, tbl_s)
    _stage_xmt(xblk, mi_v, xmt)

    @pl.loop(0, _NCH - 1, step=2)
    def _chunks(ch):
        _chunk_body(ch, ch > 0, xmt, tbl_s, bufa, sema, out_hbm, base)
        _chunk_body(ch + 1, ch > 0, xmt, tbl_s, bufb, semb, out_hbm, base)

    _chunk_body(_NCH - 1, None, xmt, tbl_s, bufa, sema, out_hbm, base)

    def _dst(c):
        return out_hbm.at[pl.ds(c * _CHW, _CHW), pl.ds(base, _RPW)]

    pltpu.make_async_copy(bufa.at[pl.ds(0, _CHW), :], _dst(_NCH - 1), sema).wait()
    pltpu.make_async_copy(bufb.at[pl.ds(0, _CHW), :], _dst(_NCH - 2), semb).wait()


def _mask_de(x2d, mk, tbl):
    f = pl.kernel(
        _body,
        out_type=jax.ShapeDtypeStruct((_OUT, _B), jnp.float32),
        mesh=plsc.VectorSubcoreMesh(core_axis_name="c", subcore_axis_name="s",
                                    num_cores=2, num_subcores=16),
        compiler_params=pltpu.CompilerParams(needs_layout_passes=False),
        scratch_types=[
            pltpu.VMEM((_RPW, _F), jnp.float32),
            pltpu.VMEM((_F,), jnp.int32),
            pltpu.VMEM((_M + _L,), jnp.int32),
            pltpu.VMEM((_NTW,), jnp.int32),
            pltpu.VMEM((_XT,), jnp.float32),
            pltpu.VMEM((_CHW + 8, _RPW), jnp.float32),
            pltpu.VMEM((_CHW + 8, _RPW), jnp.float32),
            pltpu.SMEM((_TBL.size,), jnp.int32),
            pltpu.SemaphoreType.DMA,
            pltpu.SemaphoreType.DMA,
        ],
    )
    return f(x2d, mk, tbl)


def kernel(x, mask):
    tbl = jnp.asarray(np.pad(_TBL, (0, _NTW - _TBL.size)))
    out = _mask_de(x, mask.astype(jnp.int32), tbl)
    return out.T


# speedup vs baseline: 1.0098x; 1.0098x over previous
import numpy as np
import jax
import jax.numpy as jnp
from jax import lax
from jax.experimental import pallas as pl
from jax.experimental.pallas import tpu as pltpu
from jax.experimental.pallas import tpu_sc as plsc

_B = 4096
_F = 256
_M = 128
_NPAIR = _M * (_M + 1) // 2
_OUT = _M + _NPAIR
_NW = 32
_RPW = _B // _NW
_L = 16
_NLG = _RPW // _L
_CHW = 64
_NCH = _OUT // _CHW
_XT = (_M + 1) * _RPW + 4 * _RPW


def _part_tables():
    off = lambda i: i * _M - i * (i - 1) // 2
    seg = np.zeros(_NPAIR, np.int32)
    for i in range(_M):
        seg[off(i):off(i) + _M - i] = i
    ii = np.empty(_OUT, np.int32)
    jj = np.empty(_OUT, np.int32)
    ii[:_M] = _M
    jj[:_M] = np.arange(_M)
    for c in range(_M, _OUT):
        p = c - _M
        i = int(seg[p])
        ii[c] = i
        jj[c] = i + (p - off(i))
    c0l, n, ia, jb = [], [], [], []
    for c in range(_OUT):
        if (c == 0 or ii[c] != ii[c - 1] or jj[c] != jj[c - 1] + 1
                or c % _CHW == 0):
            c0l.append(c % _CHW)
            n.append(0)
            ia.append(int(ii[c]) * _RPW)
            jb.append(int(jj[c]) * _RPW)
        n[-1] += 1
    starts = np.cumsum([0] + n[:-1])
    cpi = np.searchsorted(starts, np.arange(_NCH) * _CHW, side="left")
    cpi = np.append(cpi, len(n)).astype(np.int32)
    assert sum(n) == _OUT and max(n) <= _CHW
    return (np.asarray(c0l, np.int32), np.asarray(n, np.int32),
            np.asarray(ia, np.int32), np.asarray(jb, np.int32), cpi)


_C0L, _N, _IA, _JB, _CPI = _part_tables()
_NP = _C0L.size
_TBL = np.concatenate([_C0L, _N, _IA, _JB, _CPI])
_NTW = ((_TBL.size + 15) // 16) * 16


def _stage_midx(mk_v, mi_v):
    off = jnp.int32(0)
    lanes = lax.iota(jnp.int32, _L)
    for c in range(_F // _L):
        chunk = mk_v[pl.ds(_L * c, _L)]
        m = chunk > 0
        plsc.store_compressed(mi_v.at[pl.ds(off, _L)], lanes + _L * c, mask=m)
        off = off + plsc.all_reduce_population_count(m)[0]


def _stage_smem(tbl_v, tbl_s):
    for t in range(_NTW // _L):
        v = tbl_v[pl.ds(_L * t, _L)]
        for k in range(_L):
            if _L * t + k < _TBL.size:
                tbl_s[_L * t + k] = v[k]


def _stage_xmt(xblk, mi_v, xmt):
    ones = jnp.full((_L,), 1.0, jnp.float32)
    for lg in range(_NLG):
        xmt[pl.ds(_M * _RPW + _L * lg, _L)] = ones
    lanes = lax.iota(jnp.int32, _L)
    brow = [lanes + _L * lg for lg in range(_NLG)]

    @pl.loop(0, _M // _L)
    def _f16(f16):
        mi = mi_v[pl.ds(_L * f16, _L)]
        base = f16 * (_L * _RPW)
        pend = {}
        for k in range(_L + 1):
            if k < _L:
                mib = jnp.full((_L,), mi[k])
                pend[k] = [plsc.load_gather(xblk, [brow[lg], mib])
                           for lg in range(_NLG)]
            if k >= 1:
                vals = pend.pop(k - 1)
                for lg in range(_NLG):
                    xmt[pl.ds(base + _RPW * (k - 1) + _L * lg, _L)] = vals[lg]


def _chunk_body(ch, wait_pred, xmt, tbl_s, buf, sem, out_hbm, colbase):
    dst = out_hbm.at[pl.ds(ch * _CHW, _CHW), pl.ds(colbase, _RPW)]
    src = buf.at[pl.ds(0, _CHW), :]

    def _waits():
        pltpu.make_async_copy(src, dst, sem).wait()

    if wait_pred is None:
        _waits()
    else:
        pl.when(wait_pred)(_waits)

    plo = tbl_s[4 * _NP + ch]
    phi = tbl_s[4 * _NP + ch + 1]

    @pl.loop(plo, phi)
    def _part(pp):
        c0 = tbl_s[pp]
        npart = tbl_s[_NP + pp]
        ia = tbl_s[2 * _NP + pp]
        jb = tbl_s[3 * _NP + pp]
        va = [xmt[pl.ds(ia + _L * lg, _L)] for lg in range(_NLG)]

        @plsc.parallel_loop(0, (npart + 3) >> 2, unroll=4)
        def _colq(kq):
            k0 = kq * 4
            for u in range(4):
                jaddr = jb + (k0 + u) * _RPW
                vb = [xmt[pl.ds(jaddr + _L * lg, _L)] for lg in range(_NLG)]
                for lg in range(_NLG):
                    buf[c0 + k0 + u, pl.ds(_L * lg, _L)] = va[lg] * vb[lg]

    pltpu.async_copy(src, dst, sem)


def _body(x_hbm, m_hbm, tbl_hbm, out_hbm, xblk, mk_v, mi_v, tbl_v, xmt, bufa,
          bufb, tbl_s, sema, semb):
    wid = lax.axis_index("s") * 2 + lax.axis_index("c")
    base = wid * _RPW
    slab = pltpu.async_copy(x_hbm.at[pl.ds(base, _RPW), :], xblk, sema)
    pltpu.sync_copy(m_hbm, mk_v)
    pltpu.sync_copy(tbl_hbm, tbl_v)
    _stage_midx(mk_v, mi_v)
    _stage_smem(tbl_v, tbl_s)
    slab.wait()
    _stage_xmt(xblk, mi_v, xmt)

    @pl.loop(0, _NCH - 1, step=2)
    def _chunks(ch):
        _chunk_body(ch, ch > 0, xmt, tbl_s, bufa, sema, out_hbm, base)
        _chunk_body(ch + 1, ch > 0, xmt, tbl_s, bufb, semb, out_hbm, base)

    _chunk_body(_NCH - 1, None, xmt, tbl_s, bufa, sema, out_hbm, base)

    def _dst(c):
        return out_hbm.at[pl.ds(c * _CHW, _CHW), pl.ds(base, _RPW)]

    pltpu.make_async_copy(bufa.at[pl.ds(0, _CHW), :], _dst(_NCH - 1), sema).wait()
    pltpu.make_async_copy(bufb.at[pl.ds(0, _CHW), :], _dst(_NCH - 2), semb).wait()


def _mask_de(x2d, mk, tbl):
    f = pl.kernel(
        _body,
        out_type=jax.ShapeDtypeStruct((_OUT, _B), jnp.float32),
        mesh=plsc.VectorSubcoreMesh(core_axis_name="c", subcore_axis_name="s",
                                    num_cores=2, num_subcores=16),
        compiler_params=pltpu.CompilerParams(needs_layout_passes=False),
        scratch_types=[
            pltpu.VMEM((_RPW, _F), jnp.float32),
            pltpu.VMEM((_F,), jnp.int32),
            pltpu.VMEM((_M + _L,), jnp.int32),
            pltpu.VMEM((_NTW,), jnp.int32),
            pltpu.VMEM((_XT,), jnp.float32),
            pltpu.VMEM((_CHW + 8, _RPW), jnp.float32),
            pltpu.VMEM((_CHW + 8, _RPW), jnp.float32),
            pltpu.SMEM((_TBL.size,), jnp.int32),
            pltpu.SemaphoreType.DMA,
            pltpu.SemaphoreType.DMA,
        ],
    )
    return f(x2d, mk, tbl)


def kernel(x, mask):
    tbl = jnp.asarray(np.pad(_TBL, (0, _NTW - _TBL.size)))
    out = _mask_de(x, mask.astype(jnp.int32), tbl)
    return out.T
